# baseline (speedup 1.0000x reference)
import jax
import jax.numpy as jnp
import numpy as np
from jax.experimental import pallas as pl
from jax.experimental.pallas import tpu as pltpu

_ANCH = np.array(
    [[10, 13], [16, 30], [33, 23], [30, 61], [62, 45], [59, 119],
     [116, 90], [156, 198], [373, 326]], dtype=np.float32)
_GRIDW = (52, 26, 13)
_OFFS = (0, 8112, 10140)
_B = 8
_NT = 20


def _sel9(idx, vals):
    out = jnp.full(idx.shape, vals[8], dtype=jnp.float32)
    for k in range(7, -1, -1):
        out = jnp.where(idx == k, jnp.float32(vals[k]), out)
    return out


def _fused_body(misc_ref, sl_ref, sm_ref, sh_ref, tgt_ref,
                lraw_ref, mraw_ref, hraw_ref, out_ref,
                gatl_ref, gatm_ref, gath_ref, sem_ref):
    b = pl.program_id(0)
    iw = misc_ref[0]
    tgt = tgt_ref[0]
    x1 = tgt[:, 0:1]
    y1 = tgt[:, 1:2]
    x2 = tgt[:, 2:3]
    y2 = tgt[:, 3:4]
    cls = tgt[:, 4:5]
    w_n = x2 - x1
    h_n = y2 - y1
    vld = (w_n > 0) & (h_n > 0)
    cxn = (x1 + x2) * 0.5
    cyn = (y1 + y2) * 0.5
    w_px = w_n * iw
    h_px = h_n * iw

    ai = jax.lax.broadcasted_iota(jnp.int32, (_NT, 9), 1)
    aw9 = _sel9(ai, _ANCH[:, 0])
    ah9 = _sel9(ai, _ANCH[:, 1])
    ainter = jnp.minimum(w_px, aw9) * jnp.minimum(h_px, ah9)
    aiou = ainter / (w_px * h_px + aw9 * ah9 - ainter + 1e-9)
    mx = jnp.max(aiou, axis=1, keepdims=True)
    astar = jnp.clip(
        jnp.min(jnp.where(aiou == mx, ai, 99), axis=1, keepdims=True), 0, 8)
    s = astar // 3
    aloc = astar % 3
    gw = jnp.where(s == 0, _GRIDW[0], jnp.where(s == 1, _GRIDW[1], _GRIDW[2]))
    off = jnp.where(s == 0, _OFFS[0], jnp.where(s == 1, _OFFS[1], _OFFS[2]))
    gwf = gw.astype(jnp.float32)
    gi = jnp.clip((cxn * gwf).astype(jnp.int32), 0, gw - 1)
    gj = jnp.clip((cyn * gwf).astype(jnp.int32), 0, gw - 1)
    n = off + (gj * gw + gi) * 3 + aloc

    raws = (lraw_ref, mraw_ref, hraw_ref)

    gats = (gatl_ref, gatm_ref, gath_ref)

    def _copy(lv, t):
        W = _GRIDW[lv]
        ch0 = aloc[t, 0] * 85
        gjc = jnp.minimum(gj[t, 0], W - 1)
        return pltpu.make_async_copy(
            raws[lv].at[b, pl.ds(ch0, 85), gjc],
            gats[lv].at[t],
            sem_ref.at[lv, t])

    for t in range(_NT):
        for lv in range(3):
            _copy(lv, t).start()

    gx1 = x1 * iw
    gy1 = y1 * iw
    gx2 = x2 * iw
    gy2 = y2 * iw
    area_g = (gx2 - gx1) * (gy2 - gy1)

    back_sum = jnp.float32(0.0)
    for level, ref in ((0, sl_ref), (1, sm_ref), (2, sh_ref)):
        W = _GRIDW[level]
        HW = W * W
        OFF = _OFFS[level]
        stride = misc_ref[1 + level]
        pos = jax.lax.broadcasted_iota(jnp.int32, (1, HW), 1)
        gxf = (pos % W).astype(jnp.float32)
        gyf = (pos // W).astype(jnp.float32)
        for a in range(3):
            txs = jax.nn.sigmoid(ref[0, a, 0:1, :])
            tys = jax.nn.sigmoid(ref[0, a, 1:2, :])
            tw = ref[0, a, 2:3, :]
            th = ref[0, a, 3:4, :]
            conf_logit = ref[0, a, 4:5, :]
            cx = (txs + gxf) * stride
            cy = (tys + gyf) * stride
            aw = float(_ANCH[3 * level + a, 0])
            ah = float(_ANCH[3 * level + a, 1])
            bw = aw * jnp.exp(jnp.clip(tw, -10.0, 10.0))
            bh = ah * jnp.exp(jnp.clip(th, -10.0, 10.0))
            bx1 = cx - bw * 0.5
            by1 = cy - bh * 0.5
            bx2 = cx + bw * 0.5
            by2 = cy + bh * 0.5
            ix1 = jnp.maximum(bx1, gx1)
            iy1 = jnp.maximum(by1, gy1)
            ix2 = jnp.minimum(bx2, gx2)
            iy2 = jnp.minimum(by2, gy2)
            inter = (jnp.maximum(ix2 - ix1, 0.0)
                     * jnp.maximum(iy2 - iy1, 0.0))
            area_b = (bx2 - bx1) * (by2 - by1)
            iou = inter / (area_b + area_g - inter + 1e-9)
            max_iou = jnp.max(iou, axis=0, keepdims=True)
            back0 = max_iou <= 0.5
            n_glob = OFF + pos * 3 + a
            eq = n == n_glob
            fore = jnp.any(eq & vld, axis=0, keepdims=True)
            conf = jnp.clip(jax.nn.sigmoid(conf_logit), 1e-7, 1.0 - 1e-7)
            term = jnp.where(back0 & jnp.logical_not(fore),
                             -jnp.log(1.0 - conf), 0.0)
            back_sum = back_sum + jnp.sum(term)

    for t in range(_NT):
        for lv in range(3):
            _copy(lv, t).wait()
    rows = []
    for t in range(_NT):
        s_t = s[t, 0]
        gi_t = gi[t, 0]
        col = jnp.zeros((85, 1), jnp.float32)
        for lv in range(3):
            W = _GRIDW[lv]
            li = jax.lax.broadcasted_iota(jnp.int32, (1, W), 1)
            msk = (li == gi_t) & (s_t == lv)
            sel = jnp.where(msk, gats[lv][t], 0.0)
            col = col + jnp.sum(sel, axis=1, keepdims=True)
        rows.append(col.T)
    comp = jnp.concatenate(rows, axis=0)

    winner = jnp.full((_NT, 1), -1, jnp.int32)
    for tp in range(_NT):
        winner = jnp.where(vld[tp:tp + 1, :] & (n == n[tp:tp + 1, :]),
                           tp, winner)
    t_iota = jax.lax.broadcasted_iota(jnp.int32, (_NT, 1), 0)
    actf = (vld & (winner == t_iota)).astype(jnp.float32)

    awm = _sel9(astar, _ANCH[:, 0])
    ahm = _sel9(astar, _ANCH[:, 1])
    tx = cxn * gwf - gi.astype(jnp.float32)
    ty = cyn * gwf - gj.astype(jnp.float32)
    twt = jnp.log(jnp.maximum(w_px, 1.0) / awm)
    tht = jnp.log(jnp.maximum(h_px, 1.0) / ahm)
    scale = 2.0 - w_n * h_n

    sig0 = jax.nn.sigmoid(comp)
    px = sig0[:, 0:1]
    py = sig0[:, 1:2]
    pw = comp[:, 2:3]
    ph = comp[:, 3:4]
    pc = sig0[:, 4:5]
    sf = scale * actf
    xy_loss = jnp.sum(sf * ((px - tx) ** 2 + (py - ty) ** 2)) * 0.5
    wh_loss = jnp.sum(sf * ((pw - twt) ** 2 + (ph - tht) ** 2)) * 0.5
    pcc = jnp.clip(pc, 1e-7, 1.0 - 1e-7)
    conf_fore = jnp.sum(actf * (-jnp.log(pcc)))
    c_iota = jax.lax.broadcasted_iota(jnp.int32, (_NT, 85), 1)
    clsp = jnp.clip(sig0, 1e-7, 1.0 - 1e-7)
    onehot = c_iota == cls.astype(jnp.int32) + 5
    chm = c_iota >= 5
    bce = -jnp.where(onehot, jnp.log(clsp), jnp.log(1.0 - clsp))
    cls_loss = jnp.sum(jnp.where(chm, bce, 0.0) * actf)

    partial = xy_loss + wh_loss + conf_fore + cls_loss + back_sum
    prev = jnp.where(b == 0, 0.0, out_ref[0, 0, 0])
    tot = prev + partial
    out_ref[0, 0, 0] = jnp.where(b == _B - 1, tot / _B, tot)


_INTERPRET = False


def kernel(l_data, m_data, h_data, targets, input_wh):
    iw_i = jnp.asarray(input_wh)
    iw_f = iw_i.astype(jnp.float32)
    strides = [(iw_i // w).astype(jnp.float32) for w in _GRIDW]
    misc = jnp.stack([iw_f] + strides)
    sls = []
    for d, w in ((l_data, _GRIDW[0]), (m_data, _GRIDW[1]), (h_data, _GRIDW[2])):
        sls.append(d.reshape(_B, 3, 85, w, w)[:, :, :5].reshape(_B, 3, 5, w * w))
    out = pl.pallas_call(
        _fused_body,
        grid=(_B,),
        in_specs=[
            pl.BlockSpec(memory_space=pltpu.SMEM),
            pl.BlockSpec((1, 3, 5, _GRIDW[0] ** 2), lambda b: (b, 0, 0, 0)),
            pl.BlockSpec((1, 3, 5, _GRIDW[1] ** 2), lambda b: (b, 0, 0, 0)),
            pl.BlockSpec((1, 3, 5, _GRIDW[2] ** 2), lambda b: (b, 0, 0, 0)),
            pl.BlockSpec((1, _NT, 5), lambda b: (b, 0, 0)),
            pl.BlockSpec(memory_space=pl.ANY),
            pl.BlockSpec(memory_space=pl.ANY),
            pl.BlockSpec(memory_space=pl.ANY),
        ],
        out_specs=pl.BlockSpec((1, 1, 1), lambda b: (0, 0, 0),
                               memory_space=pltpu.SMEM),
        out_shape=jax.ShapeDtypeStruct((1, 1, 1), jnp.float32),
        scratch_shapes=[
            pltpu.VMEM((_NT, 85, _GRIDW[0]), jnp.float32),
            pltpu.VMEM((_NT, 85, _GRIDW[1]), jnp.float32),
            pltpu.VMEM((_NT, 85, _GRIDW[2]), jnp.float32),
            pltpu.SemaphoreType.DMA((3, _NT)),
        ],
        interpret=_INTERPRET,
    )(misc, sls[0], sls[1], sls[2], targets, l_data, m_data, h_data)
    return out[0, 0, 0]


# speedup vs baseline: 1.0001x; 1.0001x over previous
import jax
import jax.numpy as jnp
import numpy as np
from jax.experimental import pallas as pl
from jax.experimental.pallas import tpu as pltpu

_ANCH = np.array(
    [[10, 13], [16, 30], [33, 23], [30, 61], [62, 45], [59, 119],
     [116, 90], [156, 198], [373, 326]], dtype=np.float32)
_GRIDW = (52, 26, 13)
_OFFS = (0, 8112, 10140)
_B = 8
_NT = 20


def _sel9(idx, vals):
    out = jnp.full(idx.shape, vals[8], dtype=jnp.float32)
    for k in range(7, -1, -1):
        out = jnp.where(idx == k, jnp.float32(vals[k]), out)
    return out


def _fused_body(misc_ref, sl_ref, sm_ref, sh_ref, tgt_ref,
                lraw_ref, mraw_ref, hraw_ref, out_ref,
                gatl_ref, gatm_ref, gath_ref, sem_ref):
    b = pl.program_id(0)
    iw = misc_ref[0]
    tgt = tgt_ref[0]
    x1 = tgt[:, 0:1]
    y1 = tgt[:, 1:2]
    x2 = tgt[:, 2:3]
    y2 = tgt[:, 3:4]
    cls = tgt[:, 4:5]
    w_n = x2 - x1
    h_n = y2 - y1
    vld = (w_n > 0) & (h_n > 0)
    cxn = (x1 + x2) * 0.5
    cyn = (y1 + y2) * 0.5
    w_px = w_n * iw
    h_px = h_n * iw

    ai = jax.lax.broadcasted_iota(jnp.int32, (_NT, 9), 1)
    aw9 = _sel9(ai, _ANCH[:, 0])
    ah9 = _sel9(ai, _ANCH[:, 1])
    ainter = jnp.minimum(w_px, aw9) * jnp.minimum(h_px, ah9)
    aiou = ainter / (w_px * h_px + aw9 * ah9 - ainter + 1e-9)
    mx = jnp.max(aiou, axis=1, keepdims=True)
    astar = jnp.clip(
        jnp.min(jnp.where(aiou == mx, ai, 99), axis=1, keepdims=True), 0, 8)
    s = astar // 3
    aloc = astar % 3
    gw = jnp.where(s == 0, _GRIDW[0], jnp.where(s == 1, _GRIDW[1], _GRIDW[2]))
    off = jnp.where(s == 0, _OFFS[0], jnp.where(s == 1, _OFFS[1], _OFFS[2]))
    gwf = gw.astype(jnp.float32)
    gi = jnp.clip((cxn * gwf).astype(jnp.int32), 0, gw - 1)
    gj = jnp.clip((cyn * gwf).astype(jnp.int32), 0, gw - 1)
    n = off + (gj * gw + gi) * 3 + aloc

    raws = (lraw_ref, mraw_ref, hraw_ref)

    gats = (gatl_ref, gatm_ref, gath_ref)

    def _copy(lv, t):
        W = _GRIDW[lv]
        ch0 = aloc[t, 0] * 85
        gjc = jnp.minimum(gj[t, 0], W - 1)
        return pltpu.make_async_copy(
            raws[lv].at[b, pl.ds(ch0, 85), gjc],
            gats[lv].at[t],
            sem_ref.at[lv, t])

    for t in range(_NT):
        for lv in range(3):
            _copy(lv, t).start()

    gx1 = x1 * iw
    gy1 = y1 * iw
    gx2 = x2 * iw
    gy2 = y2 * iw
    area_g = (gx2 - gx1) * (gy2 - gy1)

    back_sum = jnp.float32(0.0)
    for level, ref in ((0, sl_ref), (1, sm_ref), (2, sh_ref)):
        W = _GRIDW[level]
        HW = W * W
        OFF = _OFFS[level]
        stride = misc_ref[1 + level]
        pos = jax.lax.broadcasted_iota(jnp.int32, (1, HW), 1)
        gxf = (pos % W).astype(jnp.float32)
        gyf = (pos // W).astype(jnp.float32)
        for a in range(3):
            txs = jax.nn.sigmoid(ref[0, a, 0:1, :])
            tys = jax.nn.sigmoid(ref[0, a, 1:2, :])
            tw = ref[0, a, 2:3, :]
            th = ref[0, a, 3:4, :]
            conf_logit = ref[0, a, 4:5, :]
            cx = (txs + gxf) * stride
            cy = (tys + gyf) * stride
            aw = float(_ANCH[3 * level + a, 0])
            ah = float(_ANCH[3 * level + a, 1])
            bw = aw * jnp.exp(jnp.clip(tw, -10.0, 10.0))
            bh = ah * jnp.exp(jnp.clip(th, -10.0, 10.0))
            bx1 = cx - bw * 0.5
            by1 = cy - bh * 0.5
            bx2 = cx + bw * 0.5
            by2 = cy + bh * 0.5
            ix1 = jnp.maximum(bx1, gx1)
            iy1 = jnp.maximum(by1, gy1)
            ix2 = jnp.minimum(bx2, gx2)
            iy2 = jnp.minimum(by2, gy2)
            inter = (jnp.maximum(ix2 - ix1, 0.0)
                     * jnp.maximum(iy2 - iy1, 0.0))
            area_b = (bx2 - bx1) * (by2 - by1)
            iou = inter / (area_b + area_g - inter + 1e-9)
            max_iou = jnp.max(iou, axis=0, keepdims=True)
            back0 = max_iou <= 0.5
            n_glob = OFF + pos * 3 + a
            eq = n == n_glob
            fore = jnp.any(eq & vld, axis=0, keepdims=True)
            conf = jnp.clip(jax.nn.sigmoid(conf_logit), 1e-7, 1.0 - 1e-7)
            term = jnp.where(back0 & jnp.logical_not(fore),
                             -jnp.log(1.0 - conf), 0.0)
            back_sum = back_sum + jnp.sum(term)

    for t in range(_NT):
        for lv in range(3):
            _copy(lv, t).wait()
    rows = []
    for t in range(_NT):
        s_t = s[t, 0]
        gi_t = gi[t, 0]
        col = jnp.zeros((85, 1), jnp.float32)
        for lv in range(3):
            W = _GRIDW[lv]
            li = jax.lax.broadcasted_iota(jnp.int32, (1, W), 1)
            msk = (li == gi_t) & (s_t == lv)
            sel = jnp.where(msk, gats[lv][t], 0.0)
            col = col + jnp.sum(sel, axis=1, keepdims=True)
        rows.append(col.T)
    comp = jnp.concatenate(rows, axis=0)

    winner = jnp.full((_NT, 1), -1, jnp.int32)
    for tp in range(_NT):
        winner = jnp.where(vld[tp:tp + 1, :] & (n == n[tp:tp + 1, :]),
                           tp, winner)
    t_iota = jax.lax.broadcasted_iota(jnp.int32, (_NT, 1), 0)
    actf = (vld & (winner == t_iota)).astype(jnp.float32)

    awm = _sel9(astar, _ANCH[:, 0])
    ahm = _sel9(astar, _ANCH[:, 1])
    tx = cxn * gwf - gi.astype(jnp.float32)
    ty = cyn * gwf - gj.astype(jnp.float32)
    twt = jnp.log(jnp.maximum(w_px, 1.0) / awm)
    tht = jnp.log(jnp.maximum(h_px, 1.0) / ahm)
    scale = 2.0 - w_n * h_n

    sig0 = jax.nn.sigmoid(comp)
    px = sig0[:, 0:1]
    py = sig0[:, 1:2]
    pw = comp[:, 2:3]
    ph = comp[:, 3:4]
    pc = sig0[:, 4:5]
    sf = scale * actf
    xy_loss = jnp.sum(sf * ((px - tx) ** 2 + (py - ty) ** 2)) * 0.5
    wh_loss = jnp.sum(sf * ((pw - twt) ** 2 + (ph - tht) ** 2)) * 0.5
    pcc = jnp.clip(pc, 1e-7, 1.0 - 1e-7)
    conf_fore = jnp.sum(actf * (-jnp.log(pcc)))
    c_iota = jax.lax.broadcasted_iota(jnp.int32, (_NT, 85), 1)
    clsp = jnp.clip(sig0, 1e-7, 1.0 - 1e-7)
    onehot = c_iota == cls.astype(jnp.int32) + 5
    chm = c_iota >= 5
    bce = -jnp.where(onehot, jnp.log(clsp), jnp.log(1.0 - clsp))
    cls_loss = jnp.sum(jnp.where(chm, bce, 0.0) * actf)

    partial = xy_loss + wh_loss + conf_fore + cls_loss + back_sum
    prev = jnp.where(b == 0, 0.0, out_ref[0, 0, 0])
    tot = prev + partial
    out_ref[0, 0, 0] = jnp.where(b == _B - 1, tot / _B, tot)


_INTERPRET = False


def kernel(l_data, m_data, h_data, targets, input_wh):
    iw_i = jnp.asarray(input_wh)
    iw_f = iw_i.astype(jnp.float32)
    strides = [(iw_i // w).astype(jnp.float32) for w in _GRIDW]
    misc = jnp.stack([iw_f] + strides)
    sls = []
    for d, w in ((l_data, _GRIDW[0]), (m_data, _GRIDW[1]), (h_data, _GRIDW[2])):
        sl = d.reshape(_B, 3, 85, w, w)[:, :, :5]
        sl = jax.lax.optimization_barrier(sl)
        sls.append(sl.reshape(_B, 3, 5, w * w))
    out = pl.pallas_call(
        _fused_body,
        grid=(_B,),
        in_specs=[
            pl.BlockSpec(memory_space=pltpu.SMEM),
            pl.BlockSpec((1, 3, 5, _GRIDW[0] ** 2), lambda b: (b, 0, 0, 0)),
            pl.BlockSpec((1, 3, 5, _GRIDW[1] ** 2), lambda b: (b, 0, 0, 0)),
            pl.BlockSpec((1, 3, 5, _GRIDW[2] ** 2), lambda b: (b, 0, 0, 0)),
            pl.BlockSpec((1, _NT, 5), lambda b: (b, 0, 0)),
            pl.BlockSpec(memory_space=pl.ANY),
            pl.BlockSpec(memory_space=pl.ANY),
            pl.BlockSpec(memory_space=pl.ANY),
        ],
        out_specs=pl.BlockSpec((1, 1, 1), lambda b: (0, 0, 0),
                               memory_space=pltpu.SMEM),
        out_shape=jax.ShapeDtypeStruct((1, 1, 1), jnp.float32),
        scratch_shapes=[
            pltpu.VMEM((_NT, 85, _GRIDW[0]), jnp.float32),
            pltpu.VMEM((_NT, 85, _GRIDW[1]), jnp.float32),
            pltpu.VMEM((_NT, 85, _GRIDW[2]), jnp.float32),
            pltpu.SemaphoreType.DMA((3, _NT)),
        ],
        interpret=_INTERPRET,
    )(misc, sls[0], sls[1], sls[2], targets, l_data, m_data, h_data)
    return out[0, 0, 0]


# speedup vs baseline: 1.8157x; 1.8155x over previous
import jax
import jax.numpy as jnp
import numpy as np
from jax.experimental import pallas as pl
from jax.experimental.pallas import tpu as pltpu

_ANCH = np.array(
    [[10, 13], [16, 30], [33, 23], [30, 61], [62, 45], [59, 119],
     [116, 90], [156, 198], [373, 326]], dtype=np.float32)
_GRIDW = (52, 26, 13)
_OFFS = (0, 8112, 10140)
_B = 8
_NT = 20


def _sel9(idx, vals):
    out = jnp.full(idx.shape, vals[8], dtype=jnp.float32)
    for k in range(7, -1, -1):
        out = jnp.where(idx == k, jnp.float32(vals[k]), out)
    return out


def _fused_body(misc_ref, tgs_ref, sl_ref, sm_ref, sh_ref, tgt_ref,
                lraw_ref, mraw_ref, hraw_ref, out_ref,
                gatl_ref, gatm_ref, gath_ref, sem_ref):
    b = pl.program_id(0)
    iw = misc_ref[0]
    tgt = tgt_ref[0]
    x1 = tgt[:, 0:1]
    y1 = tgt[:, 1:2]
    x2 = tgt[:, 2:3]
    y2 = tgt[:, 3:4]
    cls = tgt[:, 4:5]
    w_n = x2 - x1
    h_n = y2 - y1
    vld = (w_n > 0) & (h_n > 0)
    cxn = (x1 + x2) * 0.5
    cyn = (y1 + y2) * 0.5
    w_px = w_n * iw
    h_px = h_n * iw

    ai = jax.lax.broadcasted_iota(jnp.int32, (_NT, 9), 1)
    aw9 = _sel9(ai, _ANCH[:, 0])
    ah9 = _sel9(ai, _ANCH[:, 1])
    ainter = jnp.minimum(w_px, aw9) * jnp.minimum(h_px, ah9)
    aiou = ainter / (w_px * h_px + aw9 * ah9 - ainter + 1e-9)
    mx = jnp.max(aiou, axis=1, keepdims=True)
    astar = jnp.clip(
        jnp.min(jnp.where(aiou == mx, ai, 99), axis=1, keepdims=True), 0, 8)
    s = astar // 3
    aloc = astar % 3
    gw = jnp.where(s == 0, _GRIDW[0], jnp.where(s == 1, _GRIDW[1], _GRIDW[2]))
    off = jnp.where(s == 0, _OFFS[0], jnp.where(s == 1, _OFFS[1], _OFFS[2]))
    gwf = gw.astype(jnp.float32)
    gi = jnp.clip((cxn * gwf).astype(jnp.int32), 0, gw - 1)
    gj = jnp.clip((cyn * gwf).astype(jnp.int32), 0, gw - 1)
    n = off + (gj * gw + gi) * 3 + aloc

    n_s = [n[t, 0] for t in range(_NT)]
    vldi = vld.astype(jnp.int32)
    vld_s = [vldi[t, 0] != 0 for t in range(_NT)]
    aloc_s = [aloc[t, 0] for t in range(_NT)]
    gi_s = [gi[t, 0] for t in range(_NT)]
    gj_s = [gj[t, 0] for t in range(_NT)]
    s_s = [s[t, 0] for t in range(_NT)]

    raws = (lraw_ref, mraw_ref, hraw_ref)
    gats = (gatl_ref, gatm_ref, gath_ref)

    def _copy(lv, t):
        W = _GRIDW[lv]
        ch0 = aloc_s[t] * 85
        gjc = jnp.minimum(gj_s[t], W - 1)
        return pltpu.make_async_copy(
            raws[lv].at[b, pl.ds(ch0, 85), gjc],
            gats[lv].at[t],
            sem_ref.at[lv, t])

    for t in range(_NT):
        for lv in range(3):
            _copy(lv, t).start()

    back_sum = jnp.float32(0.0)
    for level, ref in ((0, sl_ref), (1, sm_ref), (2, sh_ref)):
        W = _GRIDW[level]
        OFF = _OFFS[level]
        stride = misc_ref[1 + level]
        gxi = jax.lax.broadcasted_iota(jnp.int32, (W, W), 1)
        gyi = jax.lax.broadcasted_iota(jnp.int32, (W, W), 0)
        gxf = gxi.astype(jnp.float32)
        gyf = gyi.astype(jnp.float32)
        nbase = OFF + (gyi * W + gxi) * 3
        for a in range(3):
            txs = jax.nn.sigmoid(ref[0, 5 * a + 0])
            tys = jax.nn.sigmoid(ref[0, 5 * a + 1])
            tw = ref[0, 5 * a + 2]
            th = ref[0, 5 * a + 3]
            conf_logit = ref[0, 5 * a + 4]
            cx = (txs + gxf) * stride
            cy = (tys + gyf) * stride
            aw = float(_ANCH[3 * level + a, 0])
            ah = float(_ANCH[3 * level + a, 1])
            bw = aw * jnp.exp(jnp.clip(tw, -10.0, 10.0))
            bh = ah * jnp.exp(jnp.clip(th, -10.0, 10.0))
            bx1 = cx - bw * 0.5
            by1 = cy - bh * 0.5
            bx2 = cx + bw * 0.5
            by2 = cy + bh * 0.5
            area_b = (bx2 - bx1) * (by2 - by1)
            max_iou = jnp.full((W, W), -1.0, jnp.float32)
            for t in range(_NT):
                gx1 = tgs_ref[b, t, 0] * iw
                gy1 = tgs_ref[b, t, 1] * iw
                gx2 = tgs_ref[b, t, 2] * iw
                gy2 = tgs_ref[b, t, 3] * iw
                area_g = (gx2 - gx1) * (gy2 - gy1)
                ix1 = jnp.maximum(bx1, gx1)
                iy1 = jnp.maximum(by1, gy1)
                ix2 = jnp.minimum(bx2, gx2)
                iy2 = jnp.minimum(by2, gy2)
                inter = (jnp.maximum(ix2 - ix1, 0.0)
                         * jnp.maximum(iy2 - iy1, 0.0))
                iou = inter / (area_b + area_g - inter + 1e-9)
                max_iou = jnp.maximum(max_iou, iou)
            back0 = max_iou <= 0.5
            n_glob = nbase + a
            fore = jnp.zeros((W, W), jnp.bool_)
            for t in range(_NT):
                fore = fore | ((n_glob == n_s[t]) & vld_s[t])
            conf = jnp.clip(jax.nn.sigmoid(conf_logit), 1e-7, 1.0 - 1e-7)
            term = jnp.where(back0 & jnp.logical_not(fore),
                             -jnp.log(1.0 - conf), 0.0)
            back_sum = back_sum + jnp.sum(term)

    for t in range(_NT):
        for lv in range(3):
            _copy(lv, t).wait()
    rows = []
    for t in range(_NT):
        col = jnp.zeros((85, 1), jnp.float32)
        for lv in range(3):
            W = _GRIDW[lv]
            li = jax.lax.broadcasted_iota(jnp.int32, (1, W), 1)
            msk = (li == gi_s[t]) & (s_s[t] == lv)
            sel = jnp.where(msk, gats[lv][t], 0.0)
            col = col + jnp.sum(sel, axis=1, keepdims=True)
        rows.append(col.T)
    comp = jnp.concatenate(rows, axis=0)

    winner = jnp.full((_NT, 1), -1, jnp.int32)
    for tp in range(_NT):
        winner = jnp.where(vld[tp:tp + 1, :] & (n == n[tp:tp + 1, :]),
                           tp, winner)
    t_iota = jax.lax.broadcasted_iota(jnp.int32, (_NT, 1), 0)
    actf = (vld & (winner == t_iota)).astype(jnp.float32)

    awm = _sel9(astar, _ANCH[:, 0])
    ahm = _sel9(astar, _ANCH[:, 1])
    tx = cxn * gwf - gi.astype(jnp.float32)
    ty = cyn * gwf - gj.astype(jnp.float32)
    twt = jnp.log(jnp.maximum(w_px, 1.0) / awm)
    tht = jnp.log(jnp.maximum(h_px, 1.0) / ahm)
    scale = 2.0 - w_n * h_n

    sig0 = jax.nn.sigmoid(comp)
    px = sig0[:, 0:1]
    py = sig0[:, 1:2]
    pw = comp[:, 2:3]
    ph = comp[:, 3:4]
    pc = sig0[:, 4:5]
    sf = scale * actf
    xy_loss = jnp.sum(sf * ((px - tx) ** 2 + (py - ty) ** 2)) * 0.5
    wh_loss = jnp.sum(sf * ((pw - twt) ** 2 + (ph - tht) ** 2)) * 0.5
    pcc = jnp.clip(pc, 1e-7, 1.0 - 1e-7)
    conf_fore = jnp.sum(actf * (-jnp.log(pcc)))
    c_iota = jax.lax.broadcasted_iota(jnp.int32, (_NT, 85), 1)
    clsp = jnp.clip(sig0, 1e-7, 1.0 - 1e-7)
    onehot = c_iota == cls.astype(jnp.int32) + 5
    chm = c_iota >= 5
    bce = -jnp.where(onehot, jnp.log(clsp), jnp.log(1.0 - clsp))
    cls_loss = jnp.sum(jnp.where(chm, bce, 0.0) * actf)

    partial = xy_loss + wh_loss + conf_fore + cls_loss + back_sum
    prev = jnp.where(b == 0, 0.0, out_ref[0, 0, 0])
    tot = prev + partial
    out_ref[0, 0, 0] = jnp.where(b == _B - 1, tot / _B, tot)


_INTERPRET = False


def kernel(l_data, m_data, h_data, targets, input_wh):
    iw_i = jnp.asarray(input_wh)
    iw_f = iw_i.astype(jnp.float32)
    strides = [(iw_i // w).astype(jnp.float32) for w in _GRIDW]
    misc = jnp.stack([iw_f] + strides)
    sls = []
    for d in (l_data, m_data, h_data):
        sls.append(jnp.concatenate(
            [d[:, 85 * a:85 * a + 5] for a in range(3)], axis=1))
    out = pl.pallas_call(
        _fused_body,
        grid=(_B,),
        in_specs=[
            pl.BlockSpec(memory_space=pltpu.SMEM),
            pl.BlockSpec(memory_space=pltpu.SMEM),
            pl.BlockSpec((1, 15, _GRIDW[0], _GRIDW[0]),
                         lambda b: (b, 0, 0, 0)),
            pl.BlockSpec((1, 15, _GRIDW[1], _GRIDW[1]),
                         lambda b: (b, 0, 0, 0)),
            pl.BlockSpec((1, 15, _GRIDW[2], _GRIDW[2]),
                         lambda b: (b, 0, 0, 0)),
            pl.BlockSpec((1, _NT, 5), lambda b: (b, 0, 0)),
            pl.BlockSpec(memory_space=pl.ANY),
            pl.BlockSpec(memory_space=pl.ANY),
            pl.BlockSpec(memory_space=pl.ANY),
        ],
        out_specs=pl.BlockSpec((1, 1, 1), lambda b: (0, 0, 0),
                               memory_space=pltpu.SMEM),
        out_shape=jax.ShapeDtypeStruct((1, 1, 1), jnp.float32),
        scratch_shapes=[
            pltpu.VMEM((_NT, 85, _GRIDW[0]), jnp.float32),
            pltpu.VMEM((_NT, 85, _GRIDW[1]), jnp.float32),
            pltpu.VMEM((_NT, 85, _GRIDW[2]), jnp.float32),
            pltpu.SemaphoreType.DMA((3, _NT)),
        ],
        interpret=_INTERPRET,
    )(misc, targets, sls[0], sls[1], sls[2], targets,
      l_data, m_data, h_data)
    return out[0, 0, 0]
